# Initial kernel scaffold; baseline (speedup 1.0000x reference)
import jax
import jax.numpy as jnp
from jax.experimental import pallas as pl


def kernel(x, edge_index, W_self1, W_neigh1, b1, W_self2, W_neigh2, b2):
    raise NotImplementedError("write your pallas kernel here")


# speedup vs baseline: 2.9992x; 2.9992x over previous
import functools

import jax
import jax.numpy as jnp
from jax import lax
from jax.experimental import pallas as pl
from jax.experimental.pallas import tpu as pltpu
from jax.experimental.pallas import tpu_sc as plsc

_LANES = 16
_NW = 32
_CHUNK = 8000

_F32 = jnp.float32
_HIGH = lax.Precision.HIGHEST


@functools.cache
def _sc_agg(n, e, d, with_deg):
    cpt = d // _NW
    epw = e // _NW
    nchunks = e // _CHUNK
    assert e % _CHUNK == 0 and n % _LANES == 0 and d % _NW == 0

    mesh = plsc.VectorSubcoreMesh(core_axis_name="c", subcore_axis_name="s")
    out_type = [jax.ShapeDtypeStruct((d * n,), _F32)]
    scratch = [
        pltpu.VMEM((cpt * n,), _F32),
        pltpu.VMEM((cpt * n,), _F32),
        pltpu.VMEM((_CHUNK,), jnp.int32),
        pltpu.VMEM((_CHUNK,), jnp.int32),
    ]
    if with_deg:
        out_type.append(jax.ShapeDtypeStruct((_NW, n), _F32))
        scratch += [pltpu.VMEM((n,), _F32), pltpu.VMEM((epw,), jnp.int32)]

    def body(yT, src, dst, *rest):
        if with_deg:
            aggT, degp, slab, acc, srcv, dstv, degl, degi = rest
        else:
            aggT, slab, acc, srcv, dstv = rest
        wid = lax.axis_index("s") * 2 + lax.axis_index("c")
        base = wid * cpt * n
        pltpu.sync_copy(yT.at[pl.ds(base, cpt * n)], slab)

        zf = jnp.zeros((_LANES,), _F32)

        def zb(i, carry):
            acc[pl.ds(i * _LANES, _LANES)] = zf
            return carry
        lax.fori_loop(0, cpt * n // _LANES, zb, 0)

        def chunk_body(k, carry):
            pltpu.sync_copy(src.at[pl.ds(k * _CHUNK, _CHUNK)], srcv)
            pltpu.sync_copy(dst.at[pl.ds(k * _CHUNK, _CHUNK)], dstv)

            def gb(g, c2):
                o = g * _LANES
                vs = srcv[pl.ds(o, _LANES)]
                vd = dstv[pl.ds(o, _LANES)]
                for cc in range(cpt):
                    vals = plsc.load_gather(slab, [vs + (cc * n)])
                    plsc.addupdate_scatter(acc, [vd + (cc * n)], vals)
                return c2

            lax.fori_loop(0, _CHUNK // _LANES, gb, 0)
            return carry

        lax.fori_loop(0, nchunks, chunk_body, 0)
        pltpu.sync_copy(acc, aggT.at[pl.ds(base, cpt * n)])

        if with_deg:
            def zd(i, carry):
                degl[pl.ds(i * _LANES, _LANES)] = zf
                return carry
            lax.fori_loop(0, n // _LANES, zd, 0)
            pltpu.sync_copy(dst.at[pl.ds(wid * epw, epw)], degi)
            ones = jnp.full((_LANES,), 1.0, _F32)

            def db(g, carry):
                vd = degi[pl.ds(g * _LANES, _LANES)]
                plsc.addupdate_scatter(degl, [vd], ones)
                return carry

            lax.fori_loop(0, epw // _LANES, db, 0)
            pltpu.sync_copy(degl, degp.at[wid])

    return pl.kernel(body, out_type=tuple(out_type),
                     scratch_types=tuple(scratch), mesh=mesh,
                     compiler_params=pltpu.CompilerParams(
                         needs_layout_passes=False))


def _tc_pre(x_ref, ws_ref, wn_ref, b_ref, eye_ref, sT_ref, yT_ref):
    xT = lax.dot_general(eye_ref[...], x_ref[...], (((1,), (1,)), ((), ())),
                         preferred_element_type=_F32, precision=_HIGH)
    sT_ref[...] = lax.dot_general(ws_ref[...], xT, (((0,), (0,)), ((), ())),
                                  preferred_element_type=_F32,
                                  precision=_HIGH) + b_ref[...]
    yT_ref[...] = lax.dot_general(wn_ref[...], xT, (((0,), (0,)), ((), ())),
                                  preferred_element_type=_F32, precision=_HIGH)


def _tc_mid(sT_ref, aggT_ref, degp_ref, hT_ref):
    deg = jnp.sum(degp_ref[...], axis=0, keepdims=True)
    inv = 1.0 / jnp.maximum(deg, 1.0)
    hT_ref[...] = jnp.maximum(sT_ref[...] + aggT_ref[...] * inv, 0.0)


def _tc_out(hT_ref, aggT_ref, degp_ref, ws_ref, wn_ref, b_ref, out_ref):
    deg = jnp.sum(degp_ref[...], axis=0, keepdims=True)
    inv = 1.0 / jnp.maximum(deg, 1.0)
    z = aggT_ref[...] * inv
    out_ref[...] = (
        lax.dot_general(hT_ref[...], ws_ref[...], (((0,), (0,)), ((), ())),
                        preferred_element_type=_F32, precision=_HIGH)
        + lax.dot_general(z, wn_ref[...], (((0,), (0,)), ((), ())),
                          preferred_element_type=_F32, precision=_HIGH)
        + b_ref[...])


def kernel(x, edge_index, W_self1, W_neigh1, b1, W_self2, W_neigh2, b2):
    n, d_in = x.shape
    e = edge_index.shape[1]
    d_hid = W_self1.shape[1]
    d_out = W_self2.shape[1]
    src = edge_index[0]
    dst = edge_index[1]
    eye = jnp.eye(d_in, dtype=_F32)

    s1T, y1T = pl.pallas_call(
        _tc_pre,
        out_shape=(jax.ShapeDtypeStruct((d_hid, n), _F32),
                   jax.ShapeDtypeStruct((d_hid, n), _F32)),
    )(x, W_self1, W_neigh1, b1.reshape(d_hid, 1), eye)

    agg1T, degp = _sc_agg(n, e, d_hid, True)(y1T.reshape(-1), src, dst)
    agg1T = agg1T.reshape(d_hid, n)

    hT = pl.pallas_call(
        _tc_mid, out_shape=jax.ShapeDtypeStruct((d_hid, n), _F32),
    )(s1T, agg1T, degp)

    (agg2T,) = _sc_agg(n, e, d_hid, False)(hT.reshape(-1), src, dst)
    agg2T = agg2T.reshape(d_hid, n)

    out = pl.pallas_call(
        _tc_out, out_shape=jax.ShapeDtypeStruct((n, d_out), _F32),
    )(hT, agg2T, degp, W_self2, W_neigh2, b2.reshape(1, d_out))
    return out
